# baseline (speedup 1.0000x reference)
import functools

import jax
import jax.numpy as jnp
from jax import lax
from jax.experimental import pallas as pl
from jax.experimental.pallas import tpu as pltpu
from jax.experimental.pallas import tpu_sc as plsc

_V = 1000000
_D = 32
_B = 16384
_NC = 2
_NS = 16
_L = 16
_NW = _NC * _NS
_BPW = _B // _NW
_PACK = 128 // _D
_DRAIN = _BPW * _D * 2 + _BPW * 2


def _glove_body(cidx_hbm, oidx_hbm, cooc_hbm, wt_hbm,
                wc_hbm, wo_hbm, bc_hbm, bo_hbm, dummy_hbm, out_hbm,
                cidx_v, oidx_v, cw_v, wt_v,
                ce_v, oe_v, bcb_v, bob_v, part_v, sem):
    wid = lax.axis_index("s") * _NC + lax.axis_index("c")
    base = wid * _BPW

    pltpu.sync_copy(cidx_hbm.at[pl.ds(base, _BPW)], cidx_v)
    pltpu.sync_copy(oidx_hbm.at[pl.ds(base, _BPW)], oidx_v)
    pltpu.sync_copy(cooc_hbm.at[pl.ds(base, _BPW)], cw_v)
    pltpu.sync_copy(wt_hbm.at[pl.ds(base, _BPW)], wt_v)

    def dma_step(bk, carry):
        b0 = bk * _L
        cv = cidx_v[pl.ds(b0, _L)]
        ov = oidx_v[pl.ds(b0, _L)]
        for j in range(_L):
            p = b0 + j
            g = p >> 2
            q = (p & 3) * _D
            ps, pm = p >> 3, p & 7
            c = cv[j]
            o = ov[j]
            tc_, sc_ = c >> 3, c & 7
            to_, so_ = o >> 3, o & 7
            pltpu.async_copy(wc_hbm.at[tc_, sc_],
                             ce_v.at[g, pl.ds(q, _D)], sem)
            pltpu.async_copy(wo_hbm.at[to_, so_],
                             oe_v.at[g, pl.ds(q, _D)], sem)
            pltpu.async_copy(bc_hbm.at[tc_, sc_],
                             bcb_v.at[pm, pl.ds(ps, 1)], sem)
            pltpu.async_copy(bo_hbm.at[to_, so_],
                             bob_v.at[pm, pl.ds(ps, 1)], sem)
        return carry

    lax.fori_loop(0, _BPW // _L, dma_step, 0)

    pltpu.make_async_copy(dummy_hbm, ce_v, sem).wait()
    pltpu.make_async_copy(dummy_hbm, oe_v, sem).wait()
    pltpu.make_async_copy(
        dummy_hbm.at[pl.ds(0, 4)], bcb_v.at[pl.ds(0, 4)], sem).wait()
    pltpu.make_async_copy(
        dummy_hbm.at[pl.ds(0, 4)], bob_v.at[pl.ds(0, 4)], sem).wait()

    iota = lax.iota(jnp.int32, _L)

    def blk_step(bk, tot):
        rows = bk * _L + iota
        gvec = lax.shift_right_logical(rows, 2)
        col0 = lax.bitwise_and(rows, 3) * _D
        bsv = lax.bitwise_and(rows, 7)
        biv = lax.shift_right_logical(rows, 3)
        acc = jnp.zeros((_L,), jnp.float32)
        for d in range(_D):
            a = plsc.load_gather(ce_v, [gvec, col0 + d])
            e = plsc.load_gather(oe_v, [gvec, col0 + d])
            acc = acc + a * e
        bc16 = plsc.load_gather(bcb_v, [bsv, biv])
        bo16 = plsc.load_gather(bob_v, [bsv, biv])
        p0 = bk * _L
        cw16 = cw_v[pl.ds(p0, _L)]
        wt16 = wt_v[pl.ds(p0, _L)]
        err = acc + bc16 + bo16 - cw16
        return tot + wt16 * err * err

    tot = lax.fori_loop(0, _BPW // _L, blk_step,
                        jnp.zeros((_L,), jnp.float32))
    part_v[...] = tot
    pltpu.sync_copy(part_v, out_hbm.at[wid])


@jax.jit
def _glove(cidx1, oidx1, cooc1, wt1, wc3, wo3, bc3, bo3, dummy):
    mesh = plsc.VectorSubcoreMesh(core_axis_name="c", subcore_axis_name="s")
    run = functools.partial(
        pl.kernel,
        mesh=mesh,
        compiler_params=pltpu.CompilerParams(
            needs_layout_passes=False, use_tc_tiling_on_sc=True),
        out_type=jax.ShapeDtypeStruct((_NW, _L), jnp.float32),
        scratch_types=[
            pltpu.VMEM((_BPW,), jnp.int32),
            pltpu.VMEM((_BPW,), jnp.int32),
            pltpu.VMEM((_BPW,), jnp.float32),
            pltpu.VMEM((_BPW,), jnp.float32),
            pltpu.VMEM((_BPW // _PACK, 128), jnp.float32),
            pltpu.VMEM((_BPW // _PACK, 128), jnp.float32),
            pltpu.VMEM((8, 128), jnp.float32),
            pltpu.VMEM((8, 128), jnp.float32),
            pltpu.VMEM((_L,), jnp.float32),
            pltpu.SemaphoreType.DMA,
        ],
    )(_glove_body)
    return run(cidx1, oidx1, cooc1, wt1, wc3, wo3, bc3, bo3, dummy)


def kernel(center, outside, coocs, weighting, W_center, W_outside,
           b_center, b_outside):
    cidx1 = center.reshape(_B).astype(jnp.int32)
    oidx1 = outside.reshape(_B).astype(jnp.int32)
    cooc1 = coocs.reshape(_B)
    wt1 = weighting.reshape(_B)
    wc3 = W_center.reshape(_V // 8, 8, _D)
    wo3 = W_outside.reshape(_V // 8, 8, _D)
    bc3 = b_center.reshape(_V // 8, 8, 1)
    bo3 = b_outside.reshape(_V // 8, 8, 1)
    dummy = jnp.zeros((_BPW // _PACK, 128), jnp.float32)
    partials = _glove(cidx1, oidx1, cooc1, wt1, wc3, wo3, bc3, bo3, dummy)
    return jnp.sum(partials)


# speedup vs baseline: 1.4586x; 1.4586x over previous
import functools

import jax
import jax.numpy as jnp
from jax import lax
from jax.experimental import pallas as pl
from jax.experimental.pallas import tpu as pltpu
from jax.experimental.pallas import tpu_sc as plsc

_V = 1000000
_D = 32
_B = 16384
_NC = 2
_NS = 16
_L = 16
_NW = _NC * _NS
_BPW = _B // _NW
_CHUNK = 128
_NCHUNK = _BPW // _CHUNK
_NBLK = _BPW // _L
_BLK_PER_CHUNK = _CHUNK // _L


def _glove_body(center_hbm, outside_hbm, cooc_hbm, wt_hbm,
                wc_hbm, wo_hbm, bc_hbm, bo_hbm, out_hbm,
                cidx_v, oidx_v, ce_v, oe_v, bc_v, bo_v, cw_v, wt_v,
                part_v, sem):
    wid = lax.axis_index("s") * _NC + lax.axis_index("c")

    pltpu.sync_copy(center_hbm.at[wid], cidx_v)
    pltpu.sync_copy(outside_hbm.at[wid], oidx_v)
    pltpu.sync_copy(cooc_hbm.at[wid], cw_v)
    pltpu.sync_copy(wt_hbm.at[wid], wt_v)

    descs = []
    for j in range(_NCHUNK):
        descs.append(pltpu.async_copy(
            wc_hbm.at[cidx_v.at[j]], ce_v.at[pl.ds(j * _CHUNK, _CHUNK)], sem))
        descs.append(pltpu.async_copy(
            wo_hbm.at[oidx_v.at[j]], oe_v.at[pl.ds(j * _CHUNK, _CHUNK)], sem))
        descs.append(pltpu.async_copy(
            bc_hbm.at[cidx_v.at[j]], bc_v.at[pl.ds(j * _CHUNK, _CHUNK)], sem))
        descs.append(pltpu.async_copy(
            bo_hbm.at[oidx_v.at[j]], bo_v.at[pl.ds(j * _CHUNK, _CHUNK)], sem))
    for d in descs:
        d.wait()

    iota = lax.iota(jnp.int32, _L)

    def blk_step(blk, tot):
        rows = blk * _L + iota
        acc = jnp.zeros((_L,), jnp.float32)
        for d in range(_D):
            dvec = jnp.full((_L,), d, jnp.int32)
            a = plsc.load_gather(ce_v, [rows, dvec])
            b = plsc.load_gather(oe_v, [rows, dvec])
            acc = acc + a * b
        base = blk * _L
        bc16 = bc_v[pl.ds(base, _L)]
        bo16 = bo_v[pl.ds(base, _L)]
        cw16 = cw_v[pl.ds(base, _L)]
        wt16 = wt_v[pl.ds(base, _L)]
        err = acc + bc16 + bo16 - cw16
        return tot + wt16 * err * err

    tot = lax.fori_loop(0, _NBLK, blk_step, jnp.zeros((_L,), jnp.float32))
    part_v[...] = tot
    pltpu.sync_copy(part_v, out_hbm.at[wid])


@jax.jit
def _glove(center3, outside3, cooc2, wt2, wc, wo, bc1, bo1):
    mesh = plsc.VectorSubcoreMesh(core_axis_name="c", subcore_axis_name="s")
    run = functools.partial(
        pl.kernel,
        mesh=mesh,
        compiler_params=pltpu.CompilerParams(
            needs_layout_passes=False, use_tc_tiling_on_sc=False),
        out_type=jax.ShapeDtypeStruct((_NW, _L), jnp.float32),
        scratch_types=[
            pltpu.VMEM((_NCHUNK, _CHUNK), jnp.int32),
            pltpu.VMEM((_NCHUNK, _CHUNK), jnp.int32),
            pltpu.VMEM((_BPW, _D), jnp.float32),
            pltpu.VMEM((_BPW, _D), jnp.float32),
            pltpu.VMEM((_BPW,), jnp.float32),
            pltpu.VMEM((_BPW,), jnp.float32),
            pltpu.VMEM((_BPW,), jnp.float32),
            pltpu.VMEM((_BPW,), jnp.float32),
            pltpu.VMEM((_L,), jnp.float32),
            pltpu.SemaphoreType.DMA,
        ],
    )(_glove_body)
    return run(center3, outside3, cooc2, wt2, wc, wo, bc1, bo1)


def kernel(center, outside, coocs, weighting, W_center, W_outside,
           b_center, b_outside):
    center3 = center.T.reshape(_NW, _NCHUNK, _CHUNK).astype(jnp.int32)
    outside3 = outside.T.reshape(_NW, _NCHUNK, _CHUNK).astype(jnp.int32)
    cooc2 = coocs.T.reshape(_NW, _BPW)
    wt2 = weighting.T.reshape(_NW, _BPW)
    bc1 = b_center.T.reshape(_V)
    bo1 = b_outside.T.reshape(_V)
    partials = _glove(center3, outside3, cooc2, wt2,
                      W_center, W_outside, bc1, bo1)
    return jnp.sum(partials)
